# baseline (speedup 1.0000x reference)
import jax
import jax.numpy as jnp
from jax.experimental import pallas as pl
from jax.experimental.pallas import tpu as pltpu

_B = 512
_T = 8
_DIN = 768
_DSAE = 4096
_K = 128
_BASE = _DSAE // _T

_INTERPRET = False


_NSPLIT = 2
_NBLK = _DSAE // _NSPLIT


def _encode_body(x_ref, w_ref, b_ref, z_ref, zb_ref, acc_ref):
    kb = pl.program_id(0)
    nk = pl.num_programs(0)
    h = pl.program_id(1)

    @pl.when(kb == 0)
    def _init():
        acc_ref[:, pl.ds(h * _NBLK, _NBLK)] = jnp.zeros(
            (_B, _NBLK), jnp.float32)

    acc_ref[:, pl.ds(h * _NBLK, _NBLK)] += jnp.dot(
        x_ref[...], w_ref[0, :, :], preferred_element_type=jnp.float32)

    @pl.when((kb == nk - 1) & (h == _NSPLIT - 1))
    def _finish():
        pre = acc_ref[...] + b_ref[...]
        u = jax.lax.bitcast_convert_type(pre, jnp.uint32)
        m = jnp.where(pre < 0.0, ~u, u | jnp.uint32(0x80000000))

        def step(i, t):
            shift = (31 - i).astype(jnp.uint32)
            cand = t | (jnp.uint32(1) << shift)
            cnt = jnp.sum((m >= cand).astype(jnp.int32), axis=1,
                          keepdims=True)
            return jnp.where(cnt >= _K, cand, t)

        t0 = jnp.zeros((pre.shape[0], 1), jnp.uint32)
        thr = jax.lax.fori_loop(0, 32, step, t0)
        z = jnp.where(m >= thr, jnp.maximum(pre, 0.0), 0.0)
        z_ref[...] = z
        zb_ref[...] = z.astype(jnp.bfloat16)


def _encode(xf, w3, b2):
    return pl.pallas_call(
        _encode_body,
        grid=(_T, _NSPLIT),
        in_specs=[
            pl.BlockSpec((_B, _DIN), lambda k, h: (0, k)),
            pl.BlockSpec((1, _DIN, _NBLK), lambda k, h: (k, 0, h)),
            pl.BlockSpec((1, _DSAE), lambda k, h: (0, 0)),
        ],
        out_specs=[
            pl.BlockSpec((_B, _DSAE), lambda k, h: (0, 0)),
            pl.BlockSpec((_B, _DSAE), lambda k, h: (0, 0)),
        ],
        out_shape=[
            jax.ShapeDtypeStruct((_B, _DSAE), jnp.float32),
            jax.ShapeDtypeStruct((_B, _DSAE), jnp.bfloat16),
        ],
        scratch_shapes=[pltpu.VMEM((_B, _DSAE), jnp.float32)],
        compiler_params=pltpu.CompilerParams(
            vmem_limit_bytes=66846720),
        interpret=_INTERPRET,
    )(xf, w3, b2)


def _make_decode_body(s, with_xhat):
    start = (_T - s) // 2

    def body(z_ref, w_ref, b_ref, x_ref, *refs):
        if with_xhat:
            xhat_ref, loss_ref, acc_ref = refs
        else:
            loss_ref, acc_ref = refs
        k = pl.program_id(0)
        nk = pl.num_programs(0)

        @pl.when(k == 0)
        def _init():
            acc_ref[...] = jnp.zeros_like(acc_ref)

        zblk = z_ref[...]
        for j in range(s):
            wb = w_ref[:, j, :].astype(jnp.bfloat16)
            acc_ref[:, j * _DIN:(j + 1) * _DIN] += jnp.dot(
                zblk, wb, preferred_element_type=jnp.float32)

        @pl.when(k == nk - 1)
        def _finish():
            tot = jnp.zeros((1, 1), jnp.float32)
            for j in range(s):
                out_j = acc_ref[:, j * _DIN:(j + 1) * _DIN] + b_ref[j, :]
                if with_xhat:
                    xhat_ref[:, j, :] = out_j
                d = out_j - x_ref[:, start + j, :]
                tot += jnp.sum(d * d)
            loss_ref[0, 0] = tot[0, 0]

    return body


def _decode(zb, w3, b2, x, s, with_xhat):
    kblk = 256
    prefix = _BASE * s
    out_specs = [pl.BlockSpec(memory_space=pltpu.SMEM)]
    out_shape = [jax.ShapeDtypeStruct((1, 1), jnp.float32)]
    if with_xhat:
        out_specs.insert(0, pl.BlockSpec((_B, s, _DIN), lambda k: (0, 0, 0)))
        out_shape.insert(0, jax.ShapeDtypeStruct((_B, s, _DIN), jnp.float32))
    res = pl.pallas_call(
        _make_decode_body(s, with_xhat),
        grid=(prefix // kblk,),
        in_specs=[
            pl.BlockSpec((_B, kblk), lambda k: (0, k)),
            pl.BlockSpec((kblk, s, _DIN), lambda k: (k, 0, 0)),
            pl.BlockSpec((s, _DIN), lambda k: (0, 0)),
            pl.BlockSpec((_B, _T, _DIN), lambda k: (0, 0, 0)),
        ],
        out_specs=out_specs,
        out_shape=out_shape,
        scratch_shapes=[pltpu.VMEM((_B, s * _DIN), jnp.float32)],
        compiler_params=pltpu.CompilerParams(
            vmem_limit_bytes=66846720),
        interpret=_INTERPRET,
    )(zb, w3, b2, x)
    if with_xhat:
        return res[0], res[1]
    return None, res[0]


def kernel(x, W_enc, b_enc,
           W_dec0, W_dec1, W_dec2, W_dec3, W_dec4, W_dec5, W_dec6, W_dec7,
           b_dec0, b_dec1, b_dec2, b_dec3, b_dec4, b_dec5, b_dec6, b_dec7):
    W_decs = [W_dec0, W_dec1, W_dec2, W_dec3, W_dec4, W_dec5, W_dec6, W_dec7]
    b_decs = [b_dec0, b_dec1, b_dec2, b_dec3, b_dec4, b_dec5, b_dec6, b_dec7]

    xf = x.reshape(_B, _T * _DIN)
    z, zb = _encode(xf, W_enc, b_enc.reshape(1, _DSAE))

    losses = []
    x_hat_full = None
    for t in range(_T):
        s = t + 1
        xhat, loss = _decode(zb, W_decs[t], b_decs[t], x, s, t == _T - 1)
        losses.append(loss[0, 0] / (_B * s))
        if t == _T - 1:
            x_hat_full = xhat

    total_loss = jnp.mean(jnp.stack(losses))
    return total_loss, x_hat_full, z


# speedup vs baseline: 1.4172x; 1.4172x over previous
import jax
import jax.numpy as jnp
from jax.experimental import pallas as pl
from jax.experimental.pallas import tpu as pltpu

_B = 512
_T = 8
_DIN = 768
_DSAE = 4096
_K = 128
_BASE = _DSAE // _T

_INTERPRET = False


_NSPLIT = 2
_NBLK = _DSAE // _NSPLIT


def _encode_body(x_ref, w_ref, b_ref, z_ref, zb_ref, acc_ref):
    kb = pl.program_id(0)
    nk = pl.num_programs(0)
    h = pl.program_id(1)

    @pl.when(kb == 0)
    def _init():
        acc_ref[:, pl.ds(h * _NBLK, _NBLK)] = jnp.zeros(
            (_B, _NBLK), jnp.float32)

    acc_ref[:, pl.ds(h * _NBLK, _NBLK)] += jnp.dot(
        x_ref[...], w_ref[0, :, :], preferred_element_type=jnp.float32)

    @pl.when((kb == nk - 1) & (h == _NSPLIT - 1))
    def _finish():
        pre = acc_ref[...] + b_ref[...]
        u = jax.lax.bitcast_convert_type(pre, jnp.uint32)
        m = jnp.where(pre < 0.0, ~u, u | jnp.uint32(0x80000000))

        def step(i, t):
            shift = (31 - i).astype(jnp.uint32)
            cand = t | (jnp.uint32(1) << shift)
            cnt = jnp.sum((m >= cand).astype(jnp.int32), axis=1,
                          keepdims=True)
            return jnp.where(cnt >= _K, cand, t)

        t0 = jnp.zeros((pre.shape[0], 1), jnp.uint32)
        thr = jax.lax.fori_loop(0, 32, step, t0)
        z = jnp.where(m >= thr, jnp.maximum(pre, 0.0), 0.0)
        z_ref[...] = z
        zb_ref[...] = z.astype(jnp.bfloat16)


def _encode(xf, w3, b2):
    return pl.pallas_call(
        _encode_body,
        grid=(_T, _NSPLIT),
        in_specs=[
            pl.BlockSpec((_B, _DIN), lambda k, h: (0, k)),
            pl.BlockSpec((1, _DIN, _NBLK), lambda k, h: (k, 0, h)),
            pl.BlockSpec((1, _DSAE), lambda k, h: (0, 0)),
        ],
        out_specs=[
            pl.BlockSpec((_B, _DSAE), lambda k, h: (0, 0)),
            pl.BlockSpec((_B, _DSAE), lambda k, h: (0, 0)),
        ],
        out_shape=[
            jax.ShapeDtypeStruct((_B, _DSAE), jnp.float32),
            jax.ShapeDtypeStruct((_B, _DSAE), jnp.bfloat16),
        ],
        scratch_shapes=[pltpu.VMEM((_B, _DSAE), jnp.float32)],
        compiler_params=pltpu.CompilerParams(
            vmem_limit_bytes=66846720),
        interpret=_INTERPRET,
    )(xf, w3, b2)


def _make_decode_body(s, with_xhat):
    start = (_T - s) // 2

    def body(z_ref, w_ref, b_ref, x_ref, *refs):
        if with_xhat:
            xhat_ref, loss_ref, acc_ref = refs
        else:
            loss_ref, acc_ref = refs
        k = pl.program_id(0)
        nk = pl.num_programs(0)

        @pl.when(k == 0)
        def _init():
            acc_ref[...] = jnp.zeros_like(acc_ref)

        zblk = z_ref[...]
        for j in range(s):
            wb = w_ref[:, j, :].astype(jnp.bfloat16)
            acc_ref[:, j * _DIN:(j + 1) * _DIN] += jnp.dot(
                zblk, wb, preferred_element_type=jnp.float32)

        @pl.when(k == nk - 1)
        def _finish():
            tot = jnp.zeros((1, 1), jnp.float32)
            for j in range(s):
                out_j = acc_ref[:, j * _DIN:(j + 1) * _DIN] + b_ref[j, :]
                if with_xhat:
                    xhat_ref[:, j, :] = out_j
                d = out_j - x_ref[:, start + j, :]
                tot += jnp.sum(d * d)
            loss_ref[0, 0] = tot[0, 0]

    return body


def _make_loss_body(with_xhat):
    def body(z_ref, w_ref, b_ref, xc_ref, *refs):
        if with_xhat:
            xhat_ref, loss_ref, acc_ref = refs
        else:
            loss_ref, acc_ref = refs
        j = pl.program_id(0)
        nj = pl.num_programs(0)
        wb = w_ref[...].astype(jnp.bfloat16)
        out = jnp.dot(z_ref[...], wb,
                      preferred_element_type=jnp.float32) + b_ref[...]
        if with_xhat:
            xhat_ref[...] = out
        d = out - xc_ref[...]
        part = jnp.sum(d * d)

        @pl.when(j == 0)
        def _init():
            acc_ref[0, 0] = 0.0

        acc_ref[0, 0] += part

        @pl.when(j == nj - 1)
        def _finish():
            loss_ref[0, 0] = acc_ref[0, 0]

    return body


def _decode_loss(zb, wf, b2, xf, s, with_xhat=False):
    prefix = _BASE * s
    start = (_T - s) // 2
    out_specs = [pl.BlockSpec(memory_space=pltpu.SMEM)]
    out_shape = [jax.ShapeDtypeStruct((1, 1), jnp.float32)]
    if with_xhat:
        out_specs.insert(0, pl.BlockSpec((_B, _DIN), lambda j: (0, j)))
        out_shape.insert(0,
                         jax.ShapeDtypeStruct((_B, s * _DIN), jnp.float32))
    res = pl.pallas_call(
        _make_loss_body(with_xhat),
        grid=(s,),
        in_specs=[
            pl.BlockSpec((_B, prefix), lambda j: (0, 0)),
            pl.BlockSpec((prefix, _DIN), lambda j: (0, j)),
            pl.BlockSpec((1, _DIN), lambda j: (0, j)),
            pl.BlockSpec((_B, _DIN), lambda j: (0, start + j)),
        ],
        out_specs=out_specs,
        out_shape=out_shape,
        scratch_shapes=[pltpu.SMEM((1, 1), jnp.float32)],
        compiler_params=pltpu.CompilerParams(
            vmem_limit_bytes=66846720),
        interpret=_INTERPRET,
    )(zb, wf, b2, xf)
    if with_xhat:
        return res[0], res[1]
    return None, res[0]


def kernel(x, W_enc, b_enc,
           W_dec0, W_dec1, W_dec2, W_dec3, W_dec4, W_dec5, W_dec6, W_dec7,
           b_dec0, b_dec1, b_dec2, b_dec3, b_dec4, b_dec5, b_dec6, b_dec7):
    W_decs = [W_dec0, W_dec1, W_dec2, W_dec3, W_dec4, W_dec5, W_dec6, W_dec7]
    b_decs = [b_dec0, b_dec1, b_dec2, b_dec3, b_dec4, b_dec5, b_dec6, b_dec7]

    xf = x.reshape(_B, _T * _DIN)
    z, zb = _encode(xf, W_enc, b_enc.reshape(1, _DSAE))

    losses = []
    x_hat_full = None
    for t in range(_T):
        s = t + 1
        wf = W_decs[t].reshape(_BASE * s, s * _DIN)
        bf = b_decs[t].reshape(1, s * _DIN)
        xhat, loss = _decode_loss(zb, wf, bf, xf, s, t == _T - 1)
        losses.append(loss[0, 0] / (_B * s))
        if t == _T - 1:
            x_hat_full = xhat.reshape(_B, _T, _DIN)

    total_loss = jnp.mean(jnp.stack(losses))
    return total_loss, x_hat_full, z


# speedup vs baseline: 1.4176x; 1.0003x over previous
import jax
import jax.numpy as jnp
from jax.experimental import pallas as pl
from jax.experimental.pallas import tpu as pltpu

_B = 512
_T = 8
_DIN = 768
_DSAE = 4096
_K = 128
_BASE = _DSAE // _T

_INTERPRET = False


_NSPLIT = 2
_NBLK = _DSAE // _NSPLIT


def _encode_body(x_ref, w_ref, b_ref, z_ref, zb_ref, acc_ref):
    kb = pl.program_id(0)
    nk = pl.num_programs(0)
    h = pl.program_id(1)

    @pl.when(kb == 0)
    def _init():
        acc_ref[:, pl.ds(h * _NBLK, _NBLK)] = jnp.zeros(
            (_B, _NBLK), jnp.float32)

    acc_ref[:, pl.ds(h * _NBLK, _NBLK)] += jnp.dot(
        x_ref[...], w_ref[0, :, :], preferred_element_type=jnp.float32)

    @pl.when((kb == nk - 1) & (h == _NSPLIT - 1))
    def _finish():
        pre = acc_ref[...] + b_ref[...]
        u = jax.lax.bitcast_convert_type(pre, jnp.uint32)
        m = jnp.where(pre < 0.0, ~u, u | jnp.uint32(0x80000000))

        def step(i, t):
            shift = (31 - i).astype(jnp.uint32)
            cand = t | (jnp.uint32(1) << shift)
            cnt = jnp.sum((m >= cand).astype(jnp.int32), axis=1,
                          keepdims=True)
            return jnp.where(cnt >= _K, cand, t)

        t0 = jnp.zeros((pre.shape[0], 1), jnp.uint32)
        thr = jax.lax.fori_loop(0, 32, step, t0)
        z = jnp.where(m >= thr, jnp.maximum(pre, 0.0), 0.0)
        z_ref[...] = z
        zb_ref[...] = z.astype(jnp.bfloat16)


def _encode(xf, w3, b2):
    return pl.pallas_call(
        _encode_body,
        grid=(_T, _NSPLIT),
        in_specs=[
            pl.BlockSpec((_B, _DIN), lambda k, h: (0, k)),
            pl.BlockSpec((1, _DIN, _NBLK), lambda k, h: (k, 0, h)),
            pl.BlockSpec((1, _DSAE), lambda k, h: (0, 0)),
        ],
        out_specs=[
            pl.BlockSpec((_B, _DSAE), lambda k, h: (0, 0)),
            pl.BlockSpec((_B, _DSAE), lambda k, h: (0, 0)),
        ],
        out_shape=[
            jax.ShapeDtypeStruct((_B, _DSAE), jnp.float32),
            jax.ShapeDtypeStruct((_B, _DSAE), jnp.bfloat16),
        ],
        scratch_shapes=[pltpu.VMEM((_B, _DSAE), jnp.float32)],
        compiler_params=pltpu.CompilerParams(
            vmem_limit_bytes=66846720),
        interpret=_INTERPRET,
    )(xf, w3, b2)


def _make_loss_body(with_xhat):
    def body(z_ref, w_ref, b_ref, xc_ref, *refs):
        if with_xhat:
            xhat_ref, loss_ref, acc_ref = refs
        else:
            loss_ref, acc_ref = refs
        j = pl.program_id(0)
        nj = pl.num_programs(0)
        wb = w_ref[...].astype(jnp.bfloat16)
        out = jnp.dot(z_ref[...], wb,
                      preferred_element_type=jnp.float32) + b_ref[...]
        if with_xhat:
            xhat_ref[...] = out
        d = out - xc_ref[...]
        part = jnp.sum(d * d)

        @pl.when(j == 0)
        def _init():
            acc_ref[0, 0] = 0.0

        acc_ref[0, 0] += part

        @pl.when(j == nj - 1)
        def _finish():
            loss_ref[0, 0] = acc_ref[0, 0]

    return body


def _decode_loss(zb, wf, b2, xf, s, with_xhat=False):
    prefix = _BASE * s
    start = (_T - s) // 2
    out_specs = [pl.BlockSpec(memory_space=pltpu.SMEM)]
    out_shape = [jax.ShapeDtypeStruct((1, 1), jnp.float32)]
    if with_xhat:
        out_specs.insert(0, pl.BlockSpec((_B, _DIN), lambda j: (0, j)))
        out_shape.insert(0,
                         jax.ShapeDtypeStruct((_B, s * _DIN), jnp.float32))
    res = pl.pallas_call(
        _make_loss_body(with_xhat),
        grid=(s,),
        in_specs=[
            pl.BlockSpec((_B, prefix), lambda j: (0, 0)),
            pl.BlockSpec((prefix, _DIN), lambda j: (0, j)),
            pl.BlockSpec((1, _DIN), lambda j: (0, j)),
            pl.BlockSpec((_B, _DIN), lambda j: (0, start + j)),
        ],
        out_specs=out_specs,
        out_shape=out_shape,
        scratch_shapes=[pltpu.SMEM((1, 1), jnp.float32)],
        compiler_params=pltpu.CompilerParams(
            vmem_limit_bytes=66846720),
        interpret=_INTERPRET,
    )(zb, wf, b2, xf)
    if with_xhat:
        return res[0], res[1]
    return None, res[0]


def kernel(x, W_enc, b_enc,
           W_dec0, W_dec1, W_dec2, W_dec3, W_dec4, W_dec5, W_dec6, W_dec7,
           b_dec0, b_dec1, b_dec2, b_dec3, b_dec4, b_dec5, b_dec6, b_dec7):
    W_decs = [W_dec0, W_dec1, W_dec2, W_dec3, W_dec4, W_dec5, W_dec6, W_dec7]
    b_decs = [b_dec0, b_dec1, b_dec2, b_dec3, b_dec4, b_dec5, b_dec6, b_dec7]

    xf = x.reshape(_B, _T * _DIN)
    z, zb = _encode(xf, W_enc, b_enc.reshape(1, _DSAE))

    losses = []
    x_hat_full = None
    for t in range(_T):
        s = t + 1
        wf = W_decs[t].reshape(_BASE * s, s * _DIN)
        bf = b_decs[t].reshape(1, s * _DIN)
        xhat, loss = _decode_loss(zb, wf, bf, xf, s, t == _T - 1)
        losses.append(loss[0, 0] / (_B * s))
        if t == _T - 1:
            x_hat_full = xhat.reshape(_B, _T, _DIN)

    total_loss = jnp.mean(jnp.stack(losses))
    return total_loss, x_hat_full, z
